# baseline (speedup 1.0000x reference)
import functools

import jax
import jax.numpy as jnp
from jax import lax
from jax.experimental import pallas as pl
from jax.experimental.pallas import tpu as pltpu

_BLK_M = 256
_BLK_K = 4096


def _mm_kernel(a_ref, b_ref, o_ref, *, nsteps, tail):
    k = pl.program_id(1)

    @pl.when(k == 0)
    def _init():
        o_ref[...] = jnp.zeros_like(o_ref)

    @pl.when(k < nsteps - 1)
    def _body():
        o_ref[...] += jnp.dot(
            a_ref[...].astype(jnp.bfloat16),
            b_ref[...].astype(jnp.bfloat16),
            preferred_element_type=jnp.float32,
        )

    @pl.when(k == nsteps - 1)
    def _tail():
        a = a_ref[...]
        b = b_ref[...]
        col = lax.broadcasted_iota(jnp.int32, a.shape, 1)
        a = jnp.where(col < tail, a, 0.0)
        row = lax.broadcasted_iota(jnp.int32, b.shape, 0)
        b = jnp.where(row < tail, b, 0.0)
        o_ref[...] += jnp.dot(
            a.astype(jnp.bfloat16),
            b.astype(jnp.bfloat16),
            preferred_element_type=jnp.float32,
        )


def kernel(attention_weights, content_matrix):
    m, k_dim = attention_weights.shape
    _, n = content_matrix.shape
    nsteps = pl.cdiv(k_dim, _BLK_K)
    tail = k_dim - (nsteps - 1) * _BLK_K
    body = functools.partial(_mm_kernel, nsteps=nsteps, tail=tail)
    return pl.pallas_call(
        body,
        grid=(m // _BLK_M, nsteps),
        in_specs=[
            pl.BlockSpec((_BLK_M, _BLK_K), lambda i, k: (i, k)),
            pl.BlockSpec((_BLK_K, n), lambda i, k: (k, 0)),
        ],
        out_specs=pl.BlockSpec((_BLK_M, n), lambda i, k: (i, 0)),
        out_shape=jax.ShapeDtypeStruct((m, n), jnp.float32),
        compiler_params=pltpu.CompilerParams(
            dimension_semantics=("parallel", "arbitrary")
        ),
    )(attention_weights, content_matrix)


# speedup vs baseline: 1.0900x; 1.0900x over previous
import functools

import jax
import jax.numpy as jnp
from jax import lax
from jax.experimental import pallas as pl
from jax.experimental.pallas import tpu as pltpu

_BLK_K = 2048
_NBUF = 3
_NSPLIT = 4


def _copies(a_hbm, b_hbm, abuf, bbuf, sems, j, slot, m, width):
    rows = m // _NSPLIT
    cps = [
        pltpu.make_async_copy(
            a_hbm.at[pl.ds(s * rows, rows), pl.ds(j * _BLK_K, _BLK_K)],
            abuf.at[slot, pl.ds(s * rows, rows), :],
            sems.at[slot, s],
        )
        for s in range(_NSPLIT)
    ]
    cps.append(
        pltpu.make_async_copy(
            b_hbm.at[pl.ds(j * _BLK_K, width), :],
            bbuf.at[slot, pl.ds(0, width), :],
            sems.at[slot, _NSPLIT],
        )
    )
    return cps


def _mm_kernel(
    a_hbm, b_hbm, atail_ref, o_ref, abuf, bbuf, sems, *, nchunks, tail
):
    m, n = o_ref.shape

    for j in range(_NBUF - 1):
        for c in _copies(a_hbm, b_hbm, abuf, bbuf, sems, j, j, m, _BLK_K):
            c.start()

    def body(j, acc):
        nxt = j + _NBUF - 1

        @pl.when(nxt < nchunks)
        def _prefetch():
            for c in _copies(
                a_hbm, b_hbm, abuf, bbuf, sems, nxt, lax.rem(nxt, _NBUF), m,
                _BLK_K,
            ):
                c.start()

        slot = lax.rem(j, _NBUF)
        for c in _copies(a_hbm, b_hbm, abuf, bbuf, sems, j, slot, m, _BLK_K):
            c.wait()
        return acc + jnp.dot(
            abuf[slot].astype(jnp.bfloat16),
            bbuf[slot].astype(jnp.bfloat16),
            preferred_element_type=jnp.float32,
        )

    tail_cp = pltpu.make_async_copy(
        b_hbm.at[pl.ds(nchunks * _BLK_K, tail), :],
        bbuf.at[_NBUF, pl.ds(0, tail), :],
        sems.at[_NBUF, 0],
    )
    tail_cp.start()

    acc = lax.fori_loop(0, nchunks, body, jnp.zeros((m, n), jnp.float32))

    a_t = atail_ref[...]
    col = lax.broadcasted_iota(jnp.int32, a_t.shape, 1)
    a_t = jnp.where(col < tail, a_t, 0.0)
    tail_cp.wait()
    b_t = bbuf[_NBUF]
    row = lax.broadcasted_iota(jnp.int32, b_t.shape, 0)
    b_t = jnp.where(row < tail, b_t, 0.0)
    acc += jnp.dot(
        a_t.astype(jnp.bfloat16),
        b_t.astype(jnp.bfloat16),
        preferred_element_type=jnp.float32,
    )
    o_ref[...] = acc


def kernel(attention_weights, content_matrix):
    m, k_dim = attention_weights.shape
    _, n = content_matrix.shape
    nchunks = k_dim // _BLK_K
    tail = k_dim - nchunks * _BLK_K
    body = functools.partial(_mm_kernel, nchunks=nchunks, tail=tail)
    return pl.pallas_call(
        body,
        grid=(1,),
        in_specs=[
            pl.BlockSpec(memory_space=pltpu.MemorySpace.HBM),
            pl.BlockSpec(memory_space=pltpu.MemorySpace.HBM),
            pl.BlockSpec((m, _BLK_K), lambda i: (0, nchunks)),
        ],
        out_specs=pl.BlockSpec((m, n), lambda i: (0, 0)),
        out_shape=jax.ShapeDtypeStruct((m, n), jnp.float32),
        scratch_shapes=[
            pltpu.VMEM((_NBUF, m, _BLK_K), jnp.float32),
            pltpu.VMEM((_NBUF + 1, _BLK_K, n), jnp.float32),
            pltpu.SemaphoreType.DMA((_NBUF + 1, _NSPLIT + 1)),
        ],
    )(attention_weights, content_matrix, attention_weights)


# speedup vs baseline: 4.6049x; 4.2245x over previous
import functools

import jax
import jax.numpy as jnp
from jax import lax
from jax.experimental import pallas as pl
from jax.experimental.pallas import tpu as pltpu

_BLK_K = 2048


def _mm_kernel(bt_ref, at_ref, o_ref, *, nsteps, tail):
    k = pl.program_id(0)

    @pl.when(k == 0)
    def _init():
        o_ref[...] = jnp.zeros_like(o_ref)

    @pl.when(k < nsteps - 1)
    def _body():
        o_ref[...] += jnp.dot(
            bt_ref[...].astype(jnp.bfloat16),
            at_ref[...].astype(jnp.bfloat16),
            preferred_element_type=jnp.float32,
        )

    @pl.when(k == nsteps - 1)
    def _tail():
        bt = bt_ref[...]
        col = lax.broadcasted_iota(jnp.int32, bt.shape, 1)
        bt = jnp.where(col < tail, bt, 0.0)
        at = at_ref[...]
        row = lax.broadcasted_iota(jnp.int32, at.shape, 0)
        at = jnp.where(row < tail, at, 0.0)
        o_ref[...] += jnp.dot(
            bt.astype(jnp.bfloat16),
            at.astype(jnp.bfloat16),
            preferred_element_type=jnp.float32,
        )


def kernel(attention_weights, content_matrix):
    m, k_dim = attention_weights.shape
    _, n = content_matrix.shape
    at = attention_weights.T
    bt = content_matrix.T
    nsteps = pl.cdiv(k_dim, _BLK_K)
    tail = k_dim - (nsteps - 1) * _BLK_K
    body = functools.partial(_mm_kernel, nsteps=nsteps, tail=tail)
    out_t = pl.pallas_call(
        body,
        grid=(nsteps,),
        in_specs=[
            pl.BlockSpec((n, _BLK_K), lambda k: (0, k)),
            pl.BlockSpec((_BLK_K, m), lambda k: (k, 0)),
        ],
        out_specs=pl.BlockSpec((n, m), lambda k: (0, 0)),
        out_shape=jax.ShapeDtypeStruct((n, m), jnp.float32),
        compiler_params=pltpu.CompilerParams(
            dimension_semantics=("arbitrary",)
        ),
    )(bt, at)
    return out_t.T
